# baseline (speedup 1.0000x reference)
import functools

import jax
import jax.numpy as jnp
from jax import lax
from jax.experimental import pallas as pl
from jax.experimental.pallas import tpu as pltpu
from jax.experimental.pallas import tpu_sc as plsc

N = 10000
E = 320000
D = 128
L = 3
BN_EPS = 1e-5

NC = 2
NS = 16
NW = NC * NS
EPT = E // NW
K = 40
CH = EPT // K
NBUF = 5
NP = 10240
RPT = NP // NS


RB = 5
IBN = 10
ZR = 64


def _sc_agg_body(h_hbm, ei_hbm, out_hbm,
                 ibuf, rows_v, zbuf, agg_sh, isem, gsem, ssem, zsem):
    cid = lax.axis_index("c")
    sid = lax.axis_index("s")
    wid = cid * NS + sid
    rs = sid * RPT

    def _idx_start(c, s10):
        base = pl.multiple_of(wid * EPT + c * K, 8)
        pltpu.async_copy(ei_hbm.at[pl.ds(base, K)], ibuf.at[s10, 0],
                         isem.at[s10])
        pltpu.async_copy(ei_hbm.at[pl.ds(E + base, K)], ibuf.at[s10, 1],
                         isem.at[s10])

    def _idx_wait(s10):
        pltpu.make_async_copy(ei_hbm.at[pl.ds(0, K)], ibuf.at[s10, 0],
                              isem.at[s10]).wait()
        pltpu.make_async_copy(ei_hbm.at[pl.ds(0, K)], ibuf.at[s10, 1],
                              isem.at[s10]).wait()

    def _gather_start(s5, s10):
        pltpu.async_copy(h_hbm.at[ibuf.at[s10, 0, pl.ds(0, 24)]],
                         rows_v.at[s5, pl.ds(0, 24)], gsem.at[s5])
        pltpu.async_copy(h_hbm.at[ibuf.at[s10, 0, pl.ds(24, 16)]],
                         rows_v.at[s5, pl.ds(24, 16)], gsem.at[s5])

    def _gather_wait(s5):
        pltpu.make_async_copy(h_hbm.at[pl.ds(0, 24)],
                              rows_v.at[s5, pl.ds(0, 24)],
                              gsem.at[s5]).wait()
        pltpu.make_async_copy(h_hbm.at[pl.ds(0, 16)],
                              rows_v.at[s5, pl.ds(24, 16)],
                              gsem.at[s5]).wait()

    def _scatter_start(s5, s10):
        pltpu.async_copy(rows_v.at[s5], agg_sh.at[ibuf.at[s10, 1]],
                         ssem.at[s5], add=True)

    def _scatter_wait(s5):
        pltpu.make_async_copy(rows_v.at[s5], agg_sh.at[pl.ds(0, K)],
                              ssem.at[s5]).wait()

    zv = jnp.zeros((16,), jnp.float32)
    for r in range(ZR):
        for c16 in range(D // 16):
            zbuf[r, pl.ds(c16 * 16, 16)] = zv

    for j in range(8):
        _idx_start(j, j)
    for j in range(3):
        _idx_wait(j)
        _gather_start(j, j)

    for q in range(RPT // ZR):
        pltpu.async_copy(zbuf, agg_sh.at[pl.ds(rs + q * ZR, ZR)], zsem)
    for q in range(RPT // ZR):
        pltpu.make_async_copy(zbuf, agg_sh.at[pl.ds(rs, ZR)], zsem).wait()
    plsc.subcore_barrier()

    for c in range(2):
        _gather_wait(c % RB)
        _scatter_start(c % RB, c % IBN)
        _idx_start(c + 8, (c + 8) % IBN)
        _idx_wait((c + 3) % IBN)
        _gather_start((c + 3) % RB, (c + 3) % IBN)

    @pl.loop(2, CH - 8, step=IBN)
    def _(cc):
        for b in range(IBN):
            c = cc + b
            s5, s10 = (2 + b) % RB, (2 + b) % IBN
            _gather_wait(s5)
            _scatter_start(s5, s10)
            _scatter_wait((s5 + 3) % RB)
            _idx_start(c + 8, (s10 + 8) % IBN)
            _idx_wait((s10 + 3) % IBN)
            _gather_start((s5 + 3) % RB, (s10 + 3) % IBN)

    for c in range(CH - 8, CH):
        _gather_wait(c % RB)
        _scatter_start(c % RB, c % IBN)
        _scatter_wait((c + 3) % RB)
        if c + 3 < CH:
            _idx_wait((c + 3) % IBN)
            _gather_start((c + 3) % RB, (c + 3) % IBN)
    for c in range(CH - 2, CH):
        _scatter_wait(c % RB)

    plsc.subcore_barrier()
    pltpu.sync_copy(agg_sh.at[pl.ds(rs, RPT)],
                    out_hbm.at[cid, pl.ds(rs, RPT)])


@functools.lru_cache(maxsize=None)
def _sc_aggregate():
  return pl.kernel(
    _sc_agg_body,
    out_type=jax.ShapeDtypeStruct((NC, NP, D), jnp.float32),
    mesh=plsc.VectorSubcoreMesh(core_axis_name="c", subcore_axis_name="s",
                                num_cores=NC, num_subcores=NS),
    scratch_types=[
        pltpu.VMEM((IBN, 2, K), jnp.int32),
        pltpu.VMEM((RB, K, D), jnp.float32),
        pltpu.VMEM((ZR, D), jnp.float32),
        pltpu.VMEM_SHARED((NP, D), jnp.float32),
        pltpu.SemaphoreType.DMA((IBN,)),
        pltpu.SemaphoreType.DMA((RB,)),
        pltpu.SemaphoreType.DMA((RB,)),
        pltpu.SemaphoreType.DMA,
    ],
  )


BR = 1000
NBLK = N // BR


def _mlp_body(last, h_any, p_any, w1_ref, b1_ref, g1_ref, be1_ref,
              w2_ref, b2_ref, g2_ref, be2_ref, go_ref, bo_ref, out_any,
              yv, tv, hb, p0b, p1b, ob, insem, outsem):
    def in_start(i, s):
        pltpu.async_copy(h_any.at[pl.ds(i * BR, BR)], hb.at[s], insem.at[s])
        pltpu.async_copy(p_any.at[0, pl.ds(i * BR, BR)], p0b.at[s],
                         insem.at[s])
        pltpu.async_copy(p_any.at[1, pl.ds(i * BR, BR)], p1b.at[s],
                         insem.at[s])

    def in_wait(s):
        for buf in (hb, p0b, p1b):
            pltpu.make_async_copy(h_any.at[pl.ds(0, BR)], buf.at[s],
                                  insem.at[s]).wait()

    W1 = w1_ref[...]
    W2 = w2_ref[...]
    in_start(0, 0)
    s1 = jnp.zeros((D,), jnp.float32)
    q1 = jnp.zeros((D,), jnp.float32)
    for i in range(NBLK):
        if i + 1 < NBLK:
            in_start(i + 1, (i + 1) % 2)
        in_wait(i % 2)
        z = hb[i % 2] + p0b[i % 2] + p1b[i % 2]
        y = lax.dot_general(z, W1, (((1,), (1,)), ((), ())),
                            preferred_element_type=jnp.float32) + b1_ref[...]
        yv[pl.ds(i * BR, BR), :] = y
        s1 = s1 + jnp.sum(y, axis=0)
        q1 = q1 + jnp.sum(y * y, axis=0)
    m1 = s1 / N
    v1 = q1 / N - m1 * m1
    inv1 = g1_ref[...] / jnp.sqrt(v1 + BN_EPS)
    s3 = jnp.zeros((D,), jnp.float32)
    q3 = jnp.zeros((D,), jnp.float32)
    for i in range(NBLK):
        u = jax.nn.relu((yv[pl.ds(i * BR, BR), :] - m1) * inv1
                        + be1_ref[...])
        t = lax.dot_general(u, W2, (((1,), (1,)), ((), ())),
                            preferred_element_type=jnp.float32) + b2_ref[...]
        tv[pl.ds(i * BR, BR), :] = t
        s3 = s3 + jnp.sum(t, axis=0)
        q3 = q3 + jnp.sum(t * t, axis=0)
    m3 = s3 / N
    v3 = q3 / N - m3 * m3
    r3 = 1.0 / jnp.sqrt(v3 + BN_EPS)
    v5 = g2_ref[...] * g2_ref[...] * v3 * r3 * r3
    coef = go_ref[...] * g2_ref[...] * r3 / jnp.sqrt(v5 + BN_EPS)

    def out_wait(s):
        pltpu.make_async_copy(ob.at[s], out_any.at[pl.ds(0, BR)],
                              outsem.at[s]).wait()

    for i in range(NBLK):
        o = (tv[pl.ds(i * BR, BR), :] - m3) * coef + bo_ref[...]
        if not last:
            o = jax.nn.relu(o)
        if i >= 2:
            out_wait(i % 2)
        ob[i % 2] = o
        pltpu.async_copy(ob.at[i % 2], out_any.at[pl.ds(i * BR, BR)],
                         outsem.at[i % 2])
    out_wait(0)
    out_wait(1)


def _tc_mlp(h, p, w1, b1, g1, be1, w2, b2, g2, be2, go, bo, last):
    any_spec = pl.BlockSpec(memory_space=pltpu.MemorySpace.HBM)
    return pl.pallas_call(
        functools.partial(_mlp_body, last),
        out_shape=jax.ShapeDtypeStruct((N, D), jnp.float32),
        in_specs=[any_spec, any_spec] + [pl.BlockSpec()] * 10,
        out_specs=any_spec,
        scratch_shapes=[
            pltpu.VMEM((N, D), jnp.float32),
            pltpu.VMEM((N, D), jnp.float32),
            pltpu.VMEM((2, BR, D), jnp.float32),
            pltpu.VMEM((2, BR, D), jnp.float32),
            pltpu.VMEM((2, BR, D), jnp.float32),
            pltpu.VMEM((2, BR, D), jnp.float32),
            pltpu.SemaphoreType.DMA((2,)),
            pltpu.SemaphoreType.DMA((2,)),
        ],
    )(h, p, w1, b1, g1, be1, w2, b2, g2, be2, go, bo)


def kernel(x, edge_index, W1, b1, g1, be1, W2, b2, g2, be2, go, bo):
    ei = edge_index.astype(jnp.int32).reshape(2 * E)
    h = x
    for i in range(L):
        p = _sc_aggregate()(h, ei)
        h = _tc_mlp(h, p, W1[i], b1[i], g1[i], be1[i], W2[i], b2[i],
                    g2[i], be2[i], go[i], bo[i], last=(i == L - 1))
    return h


# speedup vs baseline: 1.0030x; 1.0030x over previous
import functools

import jax
import jax.numpy as jnp
from jax import lax
from jax.experimental import pallas as pl
from jax.experimental.pallas import tpu as pltpu
from jax.experimental.pallas import tpu_sc as plsc

N = 10000
E = 320000
D = 128
L = 3
BN_EPS = 1e-5

NC = 2
NS = 16
NW = NC * NS
EPT = E // NW
K = 40
CH = EPT // K
NBUF = 5
NP = 10240
RPT = NP // NS


RB = 5
IBN = 10
ZR = 64


def _sc_agg_body(h_hbm, ei_hbm, out_hbm,
                 ibuf, rows_v, zbuf, agg_sh, isem, gsem, ssem, zsem):
    cid = lax.axis_index("c")
    sid = lax.axis_index("s")
    wid = cid * NS + sid
    rs = sid * RPT

    def _idx_start(c, s10):
        base = pl.multiple_of(wid * EPT + c * K, 8)
        pltpu.async_copy(ei_hbm.at[pl.ds(base, K)], ibuf.at[s10, 0],
                         isem.at[s10])
        pltpu.async_copy(ei_hbm.at[pl.ds(E + base, K)], ibuf.at[s10, 1],
                         isem.at[s10])

    def _idx_wait(s10):
        pltpu.make_async_copy(ei_hbm.at[pl.ds(0, K)], ibuf.at[s10, 0],
                              isem.at[s10]).wait()
        pltpu.make_async_copy(ei_hbm.at[pl.ds(0, K)], ibuf.at[s10, 1],
                              isem.at[s10]).wait()

    def _gather_start(s5, s10):
        pltpu.async_copy(h_hbm.at[ibuf.at[s10, 0]], rows_v.at[s5],
                         gsem.at[s5])

    def _gather_wait(s5):
        pltpu.make_async_copy(h_hbm.at[pl.ds(0, K)], rows_v.at[s5],
                              gsem.at[s5]).wait()

    def _scatter_start(s5, s10):
        pltpu.async_copy(rows_v.at[s5], agg_sh.at[ibuf.at[s10, 1]],
                         ssem.at[s5], add=True)

    def _scatter_wait(s5):
        pltpu.make_async_copy(rows_v.at[s5], agg_sh.at[pl.ds(0, K)],
                              ssem.at[s5]).wait()

    zv = jnp.zeros((16,), jnp.float32)
    for r in range(ZR):
        for c16 in range(D // 16):
            zbuf[r, pl.ds(c16 * 16, 16)] = zv

    for j in range(8):
        _idx_start(j, j)
    for j in range(3):
        _idx_wait(j)
        _gather_start(j, j)

    for q in range(RPT // ZR):
        pltpu.async_copy(zbuf, agg_sh.at[pl.ds(rs + q * ZR, ZR)], zsem)
    for q in range(RPT // ZR):
        pltpu.make_async_copy(zbuf, agg_sh.at[pl.ds(rs, ZR)], zsem).wait()
    plsc.subcore_barrier()

    for c in range(2):
        _gather_wait(c % RB)
        _scatter_start(c % RB, c % IBN)
        _idx_start(c + 8, (c + 8) % IBN)
        _idx_wait((c + 3) % IBN)
        _gather_start((c + 3) % RB, (c + 3) % IBN)

    @pl.loop(2, CH - 8, step=IBN)
    def _(cc):
        for b in range(IBN):
            c = cc + b
            s5, s10 = (2 + b) % RB, (2 + b) % IBN
            _gather_wait(s5)
            _scatter_start(s5, s10)
            _scatter_wait((s5 + 3) % RB)
            _idx_start(c + 8, (s10 + 8) % IBN)
            _idx_wait((s10 + 3) % IBN)
            _gather_start((s5 + 3) % RB, (s10 + 3) % IBN)

    for c in range(CH - 8, CH):
        _gather_wait(c % RB)
        _scatter_start(c % RB, c % IBN)
        _scatter_wait((c + 3) % RB)
        if c + 3 < CH:
            _idx_wait((c + 3) % IBN)
            _gather_start((c + 3) % RB, (c + 3) % IBN)
    for c in range(CH - 2, CH):
        _scatter_wait(c % RB)

    plsc.subcore_barrier()
    pltpu.sync_copy(agg_sh.at[pl.ds(rs, RPT)],
                    out_hbm.at[cid, pl.ds(rs, RPT)])


@functools.lru_cache(maxsize=None)
def _sc_aggregate():
  return pl.kernel(
    _sc_agg_body,
    out_type=jax.ShapeDtypeStruct((NC, NP, D), jnp.float32),
    mesh=plsc.VectorSubcoreMesh(core_axis_name="c", subcore_axis_name="s",
                                num_cores=NC, num_subcores=NS),
    scratch_types=[
        pltpu.VMEM((IBN, 2, K), jnp.int32),
        pltpu.VMEM((RB, K, D), jnp.float32),
        pltpu.VMEM((ZR, D), jnp.float32),
        pltpu.VMEM_SHARED((NP, D), jnp.float32),
        pltpu.SemaphoreType.DMA((IBN,)),
        pltpu.SemaphoreType.DMA((RB,)),
        pltpu.SemaphoreType.DMA((RB,)),
        pltpu.SemaphoreType.DMA,
    ],
  )


BR = 1000
NBLK = N // BR


def _mlp_body(last, h_any, p_any, w1_ref, b1_ref, g1_ref, be1_ref,
              w2_ref, b2_ref, g2_ref, be2_ref, go_ref, bo_ref, out_any,
              yv, tv, hb, p0b, p1b, ob, insem, outsem):
    def in_start(i, s):
        pltpu.async_copy(h_any.at[pl.ds(i * BR, BR)], hb.at[s], insem.at[s])
        pltpu.async_copy(p_any.at[0, pl.ds(i * BR, BR)], p0b.at[s],
                         insem.at[s])
        pltpu.async_copy(p_any.at[1, pl.ds(i * BR, BR)], p1b.at[s],
                         insem.at[s])

    def in_wait(s):
        for buf in (hb, p0b, p1b):
            pltpu.make_async_copy(h_any.at[pl.ds(0, BR)], buf.at[s],
                                  insem.at[s]).wait()

    W1 = w1_ref[...]
    W2 = w2_ref[...]
    in_start(0, 0)
    s1 = jnp.zeros((D,), jnp.float32)
    q1 = jnp.zeros((D,), jnp.float32)
    for i in range(NBLK):
        if i + 1 < NBLK:
            in_start(i + 1, (i + 1) % 2)
        in_wait(i % 2)
        z = hb[i % 2] + p0b[i % 2] + p1b[i % 2]
        y = lax.dot_general(z, W1, (((1,), (1,)), ((), ())),
                            preferred_element_type=jnp.float32) + b1_ref[...]
        yv[pl.ds(i * BR, BR), :] = y
        s1 = s1 + jnp.sum(y, axis=0)
        q1 = q1 + jnp.sum(y * y, axis=0)
    m1 = s1 / N
    v1 = q1 / N - m1 * m1
    inv1 = g1_ref[...] / jnp.sqrt(v1 + BN_EPS)
    s3 = jnp.zeros((D,), jnp.float32)
    q3 = jnp.zeros((D,), jnp.float32)
    for i in range(NBLK):
        u = jax.nn.relu((yv[pl.ds(i * BR, BR), :] - m1) * inv1
                        + be1_ref[...])
        t = lax.dot_general(u, W2, (((1,), (1,)), ((), ())),
                            preferred_element_type=jnp.float32) + b2_ref[...]
        tv[pl.ds(i * BR, BR), :] = t
        s3 = s3 + jnp.sum(t, axis=0)
        q3 = q3 + jnp.sum(t * t, axis=0)
    m3 = s3 / N
    v3 = q3 / N - m3 * m3
    r3 = 1.0 / jnp.sqrt(v3 + BN_EPS)
    v5 = g2_ref[...] * g2_ref[...] * v3 * r3 * r3
    coef = go_ref[...] * g2_ref[...] * r3 / jnp.sqrt(v5 + BN_EPS)

    def out_wait(s):
        pltpu.make_async_copy(ob.at[s], out_any.at[pl.ds(0, BR)],
                              outsem.at[s]).wait()

    for i in range(NBLK):
        o = (tv[pl.ds(i * BR, BR), :] - m3) * coef + bo_ref[...]
        if not last:
            o = jax.nn.relu(o)
        if i >= 2:
            out_wait(i % 2)
        ob[i % 2] = o
        pltpu.async_copy(ob.at[i % 2], out_any.at[pl.ds(i * BR, BR)],
                         outsem.at[i % 2])
    out_wait(0)
    out_wait(1)


def _tc_mlp(h, p, w1, b1, g1, be1, w2, b2, g2, be2, go, bo, last):
    any_spec = pl.BlockSpec(memory_space=pltpu.MemorySpace.HBM)
    return pl.pallas_call(
        functools.partial(_mlp_body, last),
        out_shape=jax.ShapeDtypeStruct((N, D), jnp.float32),
        in_specs=[any_spec, any_spec] + [pl.BlockSpec()] * 10,
        out_specs=any_spec,
        scratch_shapes=[
            pltpu.VMEM((N, D), jnp.float32),
            pltpu.VMEM((N, D), jnp.float32),
            pltpu.VMEM((2, BR, D), jnp.float32),
            pltpu.VMEM((2, BR, D), jnp.float32),
            pltpu.VMEM((2, BR, D), jnp.float32),
            pltpu.VMEM((2, BR, D), jnp.float32),
            pltpu.SemaphoreType.DMA((2,)),
            pltpu.SemaphoreType.DMA((2,)),
        ],
    )(h, p, w1, b1, g1, be1, w2, b2, g2, be2, go, bo)


def kernel(x, edge_index, W1, b1, g1, be1, W2, b2, g2, be2, go, bo):
    ei = edge_index.astype(jnp.int32).reshape(2 * E)
    h = x
    for i in range(L):
        p = _sc_aggregate()(h, ei)
        h = _tc_mlp(h, p, W1[i], b1[i], g1[i], be1[i], W2[i], b2[i],
                    g2[i], be2[i], go[i], bo[i], last=(i == L - 1))
    return h


# speedup vs baseline: 1.0033x; 1.0004x over previous
import functools

import jax
import jax.numpy as jnp
from jax import lax
from jax.experimental import pallas as pl
from jax.experimental.pallas import tpu as pltpu
from jax.experimental.pallas import tpu_sc as plsc

N = 10000
E = 320000
D = 128
L = 3
BN_EPS = 1e-5

NC = 2
NS = 16
NW = NC * NS
EPT = E // NW
K = 40
CH = EPT // K
NP = 10240
RPT = NP // NS


RB = 5
IBN = 10
ZR = 64


def _sc_agg_body(h_hbm, ei_hbm, out_hbm,
                 ibuf, rows_v, zbuf, agg_sh, isem, gsem, ssem, zsem):
    cid = lax.axis_index("c")
    sid = lax.axis_index("s")
    wid = cid * NS + sid
    rs = sid * RPT

    def _idx_start(c, s10):
        base = pl.multiple_of(wid * EPT + c * K, 8)
        pltpu.async_copy(ei_hbm.at[pl.ds(base, K)], ibuf.at[s10, 0],
                         isem.at[s10])
        pltpu.async_copy(ei_hbm.at[pl.ds(E + base, K)], ibuf.at[s10, 1],
                         isem.at[s10])

    def _idx_wait(s10):
        pltpu.make_async_copy(ei_hbm.at[pl.ds(0, K)], ibuf.at[s10, 0],
                              isem.at[s10]).wait()
        pltpu.make_async_copy(ei_hbm.at[pl.ds(0, K)], ibuf.at[s10, 1],
                              isem.at[s10]).wait()

    def _gather_start(s5, s10):
        pltpu.async_copy(h_hbm.at[ibuf.at[s10, 0]], rows_v.at[s5],
                         gsem.at[s5])

    def _gather_wait(s5):
        pltpu.make_async_copy(h_hbm.at[pl.ds(0, K)], rows_v.at[s5],
                              gsem.at[s5]).wait()

    def _scatter_start(s5, s10):
        pltpu.async_copy(rows_v.at[s5], agg_sh.at[ibuf.at[s10, 1]],
                         ssem.at[s5], add=True)

    def _scatter_wait(s5):
        pltpu.make_async_copy(rows_v.at[s5], agg_sh.at[pl.ds(0, K)],
                              ssem.at[s5]).wait()

    zv = jnp.zeros((16,), jnp.float32)
    for r in range(ZR):
        for c16 in range(D // 16):
            zbuf[r, pl.ds(c16 * 16, 16)] = zv

    for j in range(8):
        _idx_start(j, j)
    for j in range(3):
        _idx_wait(j)
        _gather_start(j, j)

    for q in range(RPT // ZR):
        pltpu.async_copy(zbuf, agg_sh.at[pl.ds(rs + q * ZR, ZR)], zsem)
    for q in range(RPT // ZR):
        pltpu.make_async_copy(zbuf, agg_sh.at[pl.ds(rs, ZR)], zsem).wait()
    plsc.subcore_barrier()

    for c in range(2):
        _gather_wait(c % RB)
        _scatter_start(c % RB, c % IBN)
        _idx_start(c + 8, (c + 8) % IBN)
        _idx_wait((c + 3) % IBN)
        _gather_start((c + 3) % RB, (c + 3) % IBN)

    @pl.loop(2, CH - 8, step=IBN)
    def _(cc):
        for b in range(IBN):
            c = cc + b
            s5, s10 = (2 + b) % RB, (2 + b) % IBN
            _gather_wait(s5)
            _scatter_start(s5, s10)
            _scatter_wait((s5 + 3) % RB)
            _idx_start(c + 8, (s10 + 8) % IBN)
            _idx_wait((s10 + 3) % IBN)
            _gather_start((s5 + 3) % RB, (s10 + 3) % IBN)

    for c in range(CH - 8, CH):
        _gather_wait(c % RB)
        _scatter_start(c % RB, c % IBN)
        _scatter_wait((c + 3) % RB)
        if c + 3 < CH:
            _idx_wait((c + 3) % IBN)
            _gather_start((c + 3) % RB, (c + 3) % IBN)
    for c in range(CH - 2, CH):
        _scatter_wait(c % RB)

    plsc.subcore_barrier()
    pltpu.sync_copy(agg_sh.at[pl.ds(rs, RPT)],
                    out_hbm.at[cid, pl.ds(rs, RPT)])


@functools.lru_cache(maxsize=None)
def _sc_aggregate():
  return pl.kernel(
    _sc_agg_body,
    out_type=jax.ShapeDtypeStruct((NC, NP, D), jnp.float32),
    mesh=plsc.VectorSubcoreMesh(core_axis_name="c", subcore_axis_name="s",
                                num_cores=NC, num_subcores=NS),
    scratch_types=[
        pltpu.VMEM((IBN, 2, K), jnp.int32),
        pltpu.VMEM((RB, K, D), jnp.float32),
        pltpu.VMEM((ZR, D), jnp.float32),
        pltpu.VMEM_SHARED((NP, D), jnp.float32),
        pltpu.SemaphoreType.DMA((IBN,)),
        pltpu.SemaphoreType.DMA((RB,)),
        pltpu.SemaphoreType.DMA((RB,)),
        pltpu.SemaphoreType.DMA,
    ],
  )


BR = 1000
NBLK = N // BR


def _mlp_body(last, h_any, p_any, w1_ref, b1_ref, g1_ref, be1_ref,
              w2_ref, b2_ref, g2_ref, be2_ref, go_ref, bo_ref, out_any,
              yv, tv, hb, p0b, p1b, ob, insem, outsem):
    def in_start(i, s):
        pltpu.async_copy(h_any.at[pl.ds(i * BR, BR)], hb.at[s], insem.at[s])
        pltpu.async_copy(p_any.at[0, pl.ds(i * BR, BR)], p0b.at[s],
                         insem.at[s])
        pltpu.async_copy(p_any.at[1, pl.ds(i * BR, BR)], p1b.at[s],
                         insem.at[s])

    def in_wait(s):
        for buf in (hb, p0b, p1b):
            pltpu.make_async_copy(h_any.at[pl.ds(0, BR)], buf.at[s],
                                  insem.at[s]).wait()

    W1 = w1_ref[...]
    W2 = w2_ref[...]
    in_start(0, 0)
    s1 = jnp.zeros((D,), jnp.float32)
    q1 = jnp.zeros((D,), jnp.float32)
    for i in range(NBLK):
        if i + 1 < NBLK:
            in_start(i + 1, (i + 1) % 2)
        in_wait(i % 2)
        z = hb[i % 2] + p0b[i % 2] + p1b[i % 2]
        y = lax.dot_general(z, W1, (((1,), (1,)), ((), ())),
                            preferred_element_type=jnp.float32) + b1_ref[...]
        yv[pl.ds(i * BR, BR), :] = y
        s1 = s1 + jnp.sum(y, axis=0)
        q1 = q1 + jnp.sum(y * y, axis=0)
    m1 = s1 / N
    v1 = q1 / N - m1 * m1
    inv1 = g1_ref[...] / jnp.sqrt(v1 + BN_EPS)
    s3 = jnp.zeros((D,), jnp.float32)
    q3 = jnp.zeros((D,), jnp.float32)
    for i in range(NBLK):
        u = jax.nn.relu((yv[pl.ds(i * BR, BR), :] - m1) * inv1
                        + be1_ref[...])
        t = lax.dot_general(u, W2, (((1,), (1,)), ((), ())),
                            preferred_element_type=jnp.float32) + b2_ref[...]
        tv[pl.ds(i * BR, BR), :] = t
        s3 = s3 + jnp.sum(t, axis=0)
        q3 = q3 + jnp.sum(t * t, axis=0)
    m3 = s3 / N
    v3 = q3 / N - m3 * m3
    r3 = 1.0 / jnp.sqrt(v3 + BN_EPS)
    v5 = g2_ref[...] * g2_ref[...] * v3 * r3 * r3
    coef = go_ref[...] * g2_ref[...] * r3 / jnp.sqrt(v5 + BN_EPS)

    def out_wait(s):
        pltpu.make_async_copy(ob.at[s], out_any.at[pl.ds(0, BR)],
                              outsem.at[s]).wait()

    for i in range(NBLK):
        o = (tv[pl.ds(i * BR, BR), :] - m3) * coef + bo_ref[...]
        if not last:
            o = jax.nn.relu(o)
        if i >= 2:
            out_wait(i % 2)
        ob[i % 2] = o
        pltpu.async_copy(ob.at[i % 2], out_any.at[pl.ds(i * BR, BR)],
                         outsem.at[i % 2])
    out_wait(0)
    out_wait(1)


def _tc_mlp(h, p, w1, b1, g1, be1, w2, b2, g2, be2, go, bo, last):
    any_spec = pl.BlockSpec(memory_space=pltpu.MemorySpace.HBM)
    return pl.pallas_call(
        functools.partial(_mlp_body, last),
        out_shape=jax.ShapeDtypeStruct((N, D), jnp.float32),
        in_specs=[any_spec, any_spec] + [pl.BlockSpec()] * 10,
        out_specs=any_spec,
        scratch_shapes=[
            pltpu.VMEM((N, D), jnp.float32),
            pltpu.VMEM((N, D), jnp.float32),
            pltpu.VMEM((2, BR, D), jnp.float32),
            pltpu.VMEM((2, BR, D), jnp.float32),
            pltpu.VMEM((2, BR, D), jnp.float32),
            pltpu.VMEM((2, BR, D), jnp.float32),
            pltpu.SemaphoreType.DMA((2,)),
            pltpu.SemaphoreType.DMA((2,)),
        ],
    )(h, p, w1, b1, g1, be1, w2, b2, g2, be2, go, bo)


def kernel(x, edge_index, W1, b1, g1, be1, W2, b2, g2, be2, go, bo):
    ei = edge_index.astype(jnp.int32).reshape(2 * E)
    h = x
    for i in range(L):
        p = _sc_aggregate()(h, ei)
        h = _tc_mlp(h, p, W1[i], b1[i], g1[i], be1[i], W2[i], b2[i],
                    g2[i], be2[i], go[i], bo[i], last=(i == L - 1))
    return h
